# baseline (speedup 1.0000x reference)
import functools
import math

import jax
import jax.numpy as jnp
from jax import lax
from jax.experimental import pallas as pl
from jax.experimental.pallas import tpu as pltpu
from jax.experimental.pallas import tpu_sc as plsc

E = 8
F = 1024
H = 2048
C = 1024
T = 1.0
B = 2048

BT = 256
NB = B // BT
S = 2 * B
BLK = 256
P = S + E * BLK
NBLK = P // BLK
NBX = NBLK + 8
NW = 32
SPW = S // NW


def _route_body(h_ref, wmu_ref, wlv_ref, bmu_ref, blv_ref,
                w_ref, dest_ref, bexp_ref, carry, idxs, ranks):
    b = pl.program_id(0)

    @pl.when(b < NB)
    def _router():
        _router_block(b, h_ref, wmu_ref, wlv_ref, bmu_ref, blv_ref,
                      w_ref, carry, idxs, ranks)

    @pl.when(b == NB)
    def _finalize():
        _finalize_block(carry, idxs, ranks, dest_ref, bexp_ref)


def _router_block(b, h_ref, wmu_ref, wlv_ref, bmu_ref, blv_ref,
                  w_ref, carry, idxs, ranks):
    h = h_ref[...]
    zer = jnp.zeros((E, F), jnp.float32)
    rhs = jnp.concatenate([
        jnp.concatenate([wmu_ref[...], zer], axis=1),
        jnp.concatenate([zer, jnp.exp(wlv_ref[...])], axis=1)], axis=0)
    lhs = jnp.concatenate([h, h * h], axis=1)
    mv = lax.dot_general(lhs, rhs, (((1,), (1,)), ((), ())),
                         preferred_element_type=jnp.float32)
    mu = mv[:, :E] + bmu_ref[...]
    var = jnp.maximum(mv[:, E:] + jnp.exp(blv_ref[...]), 1e-12)
    s = mu * lax.rsqrt(1.0 + (math.pi / 8.0) * var)
    iota_e = lax.broadcasted_iota(jnp.int32, s.shape, 1)
    m1 = jnp.max(s, axis=1, keepdims=True)
    i1 = jnp.min(jnp.where(s == m1, iota_e, E), axis=1, keepdims=True)
    s2m = jnp.where(iota_e == i1, -jnp.inf, s)
    m2 = jnp.max(s2m, axis=1, keepdims=True)
    i2 = jnp.min(jnp.where(s2m == m2, iota_e, E), axis=1, keepdims=True)
    w1 = 1.0 / (1.0 + jnp.exp((m2 - m1) / T))
    w2 = 1.0 - w1

    @pl.when(b == 0)
    def _init():
        carry[...] = jnp.zeros((1, E), jnp.float32)

    one1 = (iota_e == i1).astype(jnp.float32)
    one2 = (iota_e == i2).astype(jnp.float32)
    O = jnp.concatenate([one1, one2], axis=0)
    n2 = 2 * BT
    ir = lax.broadcasted_iota(jnp.int32, (n2, n2), 0)
    ic = lax.broadcasted_iota(jnp.int32, (n2, n2), 1)
    tri = (ic < ir).astype(jnp.float32)
    prior = lax.dot_general(tri, O, (((1,), (0,)), ((), ())),
                            preferred_element_type=jnp.float32)
    cprior = prior + carry[...]
    r1 = jnp.sum(cprior[:BT] * one1, axis=1, keepdims=True)
    r2 = jnp.sum(cprior[BT:] * one2, axis=1, keepdims=True)
    colsum = jnp.sum(O, axis=0, keepdims=True)
    carry[...] += colsum

    sl = pl.ds(b * BT, BT)
    idxs[sl, :] = jnp.concatenate([i1, i2], axis=1)
    w_ref[...] = jnp.concatenate([w1, w2], axis=1)
    ranks[sl, :] = jnp.concatenate([r1, r2], axis=1).astype(jnp.int32)


def _finalize_block(carry, idxs, ranks, dest_ref, bexp_ref):
    tot = carry[...]
    pt = jnp.floor((tot + (BLK - 1)) / BLK) * BLK
    ir8 = lax.broadcasted_iota(jnp.int32, (E, E), 0)
    ic8 = lax.broadcasted_iota(jnp.int32, (E, E), 1)
    ptb = jnp.broadcast_to(pt, (E, E))
    D = jnp.where(ir8 == ic8, ptb, 0.0)
    ptc = lax.dot_general(D, jnp.ones((E, E), jnp.float32),
                          (((1,), (0,)), ((), ())),
                          preferred_element_type=jnp.float32)
    pstart = jnp.sum(jnp.where(ir8 < ic8, ptc, 0.0), axis=0, keepdims=True)

    idx = idxs[...]
    rank = ranks[...]
    for k in range(2):
        ik = idx[:, k:k + 1]
        iota_e = lax.broadcasted_iota(jnp.int32, (B, E), 1)
        onek = (iota_e == ik).astype(jnp.float32)
        base = jnp.sum(onek * pstart, axis=1, keepdims=True)
        dest_ref[:, k:k + 1] = rank[:, k:k + 1] + base.astype(jnp.int32)

    jr = lax.broadcasted_iota(jnp.int32, (NBX, E), 0)
    starts = jnp.broadcast_to(pstart, (NBX, E))
    jstart = (jr * BLK).astype(jnp.float32)
    cnt = jnp.sum((starts <= jstart).astype(jnp.int32), axis=1, keepdims=True)
    total_padded = jnp.sum(pt)
    used = jstart[:, :1] < total_padded
    bexp_ref[...] = jnp.where(used, cnt - 1, -1)


def _gemm_body(s_ref, x_ref, w1_ref, b1_ref, w2_ref, b2_ref, ws_ref, o_ref):
    i = pl.program_id(0)

    @pl.when(s_ref[i] >= 0)
    def _compute():
        x = x_ref[...].astype(jnp.bfloat16)
        w1b = w1_ref[0].astype(jnp.bfloat16)
        hid = jnp.maximum(
            lax.dot_general(x, w1b, (((1,), (0,)), ((), ())),
                            preferred_element_type=jnp.float32) + b1_ref[0], 0.0)
        w2b = w2_ref[0].astype(jnp.bfloat16)
        o = lax.dot_general(hid.astype(jnp.bfloat16), w2b,
                            (((1,), (0,)), ((), ())),
                            preferred_element_type=jnp.float32) + b2_ref[0]
        o_ref[...] = o * ws_ref[:, 0:1]


@functools.lru_cache(maxsize=None)
def _make_dispatch():
    mesh = plsc.VectorSubcoreMesh(core_axis_name="c", subcore_axis_name="s",
                                  num_cores=2)

    @functools.partial(
        pl.kernel, mesh=mesh,
        out_type=[
            jax.ShapeDtypeStruct((P, F), jnp.float32),
            jax.ShapeDtypeStruct((P, 128), jnp.float32),
        ],
        scratch_types=[
            pltpu.VMEM((4, 32), jnp.int32),
            pltpu.VMEM((4, 32), jnp.int32),
            pltpu.VMEM((4, 32, 128), jnp.float32),
            pltpu.VMEM((32, F), jnp.float32),
            pltpu.VMEM((32, F), jnp.float32),
            pltpu.VMEM((32, F), jnp.float32),
            pltpu.SemaphoreType.DMA,
            pltpu.SemaphoreType.DMA,
            pltpu.SemaphoreType.DMA,
            pltpu.SemaphoreType.DMA,
            pltpu.SemaphoreType.DMA,
            pltpu.SemaphoreType.DMA,
            pltpu.SemaphoreType.DMA,
        ],
    )
    def dispatch(h_hbm, tok_hbm, dest_hbm, w_hbm, xs_hbm, ws_hbm, tokv, didv,
                 wv, r0, r1, r2, g0, g1, g2, s0, s1, s2, sw):
        wid = lax.axis_index("s") * 2 + lax.axis_index("c")
        pltpu.sync_copy(tok_hbm.at[wid], tokv)
        pltpu.sync_copy(dest_hbm.at[wid], didv)
        pltpu.sync_copy(w_hbm.at[wid], wv)
        bufs = [r0, r1, r2]
        gsems = [g0, g1, g2]
        ssems = [s0, s1, s2]
        gh = {}
        sh = {}
        wh = {}
        for j in range(3):
            gh[j] = pltpu.async_copy(h_hbm.at[tokv.at[j]], bufs[j], gsems[j])
        for j in range(4):
            p = j % 3
            wh[j] = pltpu.async_copy(wv.at[j], ws_hbm.at[didv.at[j]], sw)
            gh[j].wait()
            sh[j] = pltpu.async_copy(bufs[p], xs_hbm.at[didv.at[j]], ssems[p])
            if j + 3 < 4:
                sh[j].wait()
                gh[j + 3] = pltpu.async_copy(
                    h_hbm.at[tokv.at[j + 3]], bufs[p], gsems[p])
        for j in range(1, 4):
            sh[j].wait()
        for j in range(4):
            wh[j].wait()

    return dispatch


@functools.lru_cache(maxsize=None)
def _make_combine():
    mesh = plsc.VectorSubcoreMesh(core_axis_name="c", subcore_axis_name="s",
                                  num_cores=2)

    @functools.partial(
        pl.kernel, mesh=mesh,
        out_type=jax.ShapeDtypeStruct((B, C), jnp.float32),
        scratch_types=[
            pltpu.VMEM((4, 16), jnp.int32),
            pltpu.VMEM((4, 16), jnp.int32),
            pltpu.VMEM((16, C), jnp.float32),
            pltpu.VMEM((16, C), jnp.float32),
            pltpu.VMEM((16, C), jnp.float32),
            pltpu.VMEM((16, C), jnp.float32),
            pltpu.VMEM((16, C), jnp.float32),
            pltpu.VMEM((16, C), jnp.float32),
            pltpu.SemaphoreType.DMA,
            pltpu.SemaphoreType.DMA,
            pltpu.SemaphoreType.DMA,
            pltpu.SemaphoreType.DMA,
        ],
    )
    def combine(os_hbm, d0_hbm, d1_hbm, out_hbm,
                d0v, d1v, b0a, b0b, b1a, b1b, oa, ob,
                ga, gb, wsa, wsb):
        wid = lax.axis_index("s") * 2 + lax.axis_index("c")
        pltpu.sync_copy(d0_hbm.at[wid], d0v)
        pltpu.sync_copy(d1_hbm.at[wid], d1v)
        b0 = [b0a, b0b]
        b1 = [b1a, b1b]
        ob_ = [oa, ob]
        gsem = [ga, gb]
        wsem = [wsa, wsb]
        gh = {}
        wh = {}
        for j in range(2):
            gh[j] = (pltpu.async_copy(os_hbm.at[d0v.at[j]], b0[j], gsem[j]),
                     pltpu.async_copy(os_hbm.at[d1v.at[j]], b1[j], gsem[j]))
        for j in range(4):
            p = j % 2
            gh[j][0].wait()
            gh[j][1].wait()
            if j >= 2:
                wh[j - 2].wait()

            def body(c, _, p=p):
                for r in range(16):
                    sl = pl.ds(c * 16, 16)
                    ob_[p][r, sl] = b0[p][r, sl] + b1[p][r, sl]
                return 0

            lax.fori_loop(0, C // 16, body, 0)
            wh[j] = pltpu.async_copy(
                ob_[p], out_hbm.at[pl.ds(wid * 64 + j * 16, 16)], wsem[p])
            if j + 2 < 4:
                gh[j + 2] = (
                    pltpu.async_copy(os_hbm.at[d0v.at[j + 2]], b0[p], gsem[p]),
                    pltpu.async_copy(os_hbm.at[d1v.at[j + 2]], b1[p], gsem[p]))
        wh[2].wait()
        wh[3].wait()

    return combine


def kernel(h, W_mu, W_logvar, b_mu, b_logvar, W1, b1, W2, b2):
    bmu = b_mu.reshape(1, E)
    blv = b_logvar.reshape(1, E)
    b1r = b1.reshape(E, 1, H)
    b2r = b2.reshape(E, 1, C)

    wts, dest, bexp = pl.pallas_call(
        _route_body,
        grid=(NB + 1,),
        in_specs=[
            pl.BlockSpec((BT, F), lambda b: (jnp.minimum(b, NB - 1), 0)),
            pl.BlockSpec((E, F), lambda b: (0, 0)),
            pl.BlockSpec((E, F), lambda b: (0, 0)),
            pl.BlockSpec((1, E), lambda b: (0, 0)),
            pl.BlockSpec((1, E), lambda b: (0, 0)),
        ],
        out_specs=[
            pl.BlockSpec((BT, 2), lambda b: (jnp.minimum(b, NB - 1), 0)),
            pl.BlockSpec((B, 2), lambda b: (0, 0)),
            pl.BlockSpec((NBX, 1), lambda b: (0, 0)),
        ],
        out_shape=[
            jax.ShapeDtypeStruct((B, 2), jnp.float32),
            jax.ShapeDtypeStruct((B, 2), jnp.int32),
            jax.ShapeDtypeStruct((NBX, 1), jnp.int32),
        ],
        scratch_shapes=[
            pltpu.VMEM((1, E), jnp.float32),
            pltpu.VMEM((B, 2), jnp.int32),
            pltpu.VMEM((B, 2), jnp.int32),
        ],
        compiler_params=pltpu.CompilerParams(
            dimension_semantics=("arbitrary",)),
    )(h, W_mu, W_logvar, bmu, blv)

    tok3 = jnp.repeat(jnp.arange(B, dtype=jnp.int32), 2).reshape(NW, 4, 32)
    dest3 = dest.reshape(NW, 4, 32)
    wrep = jnp.broadcast_to(wts.reshape(S, 1), (S, 128)).reshape(NW, 4, 32, 128)
    x_sorted, w_sorted = _make_dispatch()(h, tok3, dest3, wrep)

    out_sorted = pl.pallas_call(
        _gemm_body,
        grid_spec=pltpu.PrefetchScalarGridSpec(
            num_scalar_prefetch=1,
            grid=(NBLK,),
            in_specs=[
                pl.BlockSpec((BLK, F), lambda i, s: (i, 0)),
                pl.BlockSpec((1, F, H),
                             lambda i, s: (jnp.where(s[i] < 0, E - 1, s[i]), 0, 0)),
                pl.BlockSpec((1, 1, H),
                             lambda i, s: (jnp.where(s[i] < 0, E - 1, s[i]), 0, 0)),
                pl.BlockSpec((1, H, C),
                             lambda i, s: (jnp.where(s[i] < 0, E - 1, s[i]), 0, 0)),
                pl.BlockSpec((1, 1, C),
                             lambda i, s: (jnp.where(s[i] < 0, E - 1, s[i]), 0, 0)),
                pl.BlockSpec((BLK, 128), lambda i, s: (i, 0)),
            ],
            out_specs=pl.BlockSpec((BLK, C), lambda i, s: (i, 0)),
        ),
        out_shape=jax.ShapeDtypeStruct((P, C), jnp.float32),
        compiler_params=pltpu.CompilerParams(
            dimension_semantics=("arbitrary",)),
    )(bexp.reshape(NBX), x_sorted, W1, b1r, W2, b2r, w_sorted)

    d0 = dest[:, 0].reshape(NW, 4, 16)
    d1 = dest[:, 1].reshape(NW, 4, 16)
    return _make_combine()(out_sorted, d0, d1)


# speedup vs baseline: 1.0153x; 1.0153x over previous
import functools
import math

import jax
import jax.numpy as jnp
from jax import lax
from jax.experimental import pallas as pl
from jax.experimental.pallas import tpu as pltpu
from jax.experimental.pallas import tpu_sc as plsc

E = 8
F = 1024
H = 2048
C = 1024
T = 1.0
B = 2048

BT = 256
NB = B // BT
S = 2 * B
BLK = 256
P = S + E * BLK
NBLK = P // BLK
NBX = NBLK + 8
NW = 32
SPW = S // NW


def _route_body(h_ref, wmu_ref, wlv_ref, bmu_ref, blv_ref,
                w_ref, dest_ref, bexp_ref, carry, idxs, ranks):
    b = pl.program_id(0)

    @pl.when(b < NB)
    def _router():
        _router_block(b, h_ref, wmu_ref, wlv_ref, bmu_ref, blv_ref,
                      w_ref, carry, idxs, ranks)

    @pl.when(b == NB)
    def _finalize():
        _finalize_block(carry, idxs, ranks, dest_ref, bexp_ref)


def _router_block(b, h_ref, wmu_ref, wlv_ref, bmu_ref, blv_ref,
                  w_ref, carry, idxs, ranks):
    h = h_ref[...]
    mu = lax.dot_general(h, wmu_ref[...], (((1,), (1,)), ((), ())),
                         preferred_element_type=jnp.float32) + bmu_ref[...]
    var = lax.dot_general(h * h, jnp.exp(wlv_ref[...]), (((1,), (1,)), ((), ())),
                          preferred_element_type=jnp.float32) + jnp.exp(blv_ref[...])
    var = jnp.maximum(var, 1e-12)
    s = mu * lax.rsqrt(1.0 + (math.pi / 8.0) * var)
    iota_e = lax.broadcasted_iota(jnp.int32, s.shape, 1)
    m1 = jnp.max(s, axis=1, keepdims=True)
    i1 = jnp.min(jnp.where(s == m1, iota_e, E), axis=1, keepdims=True)
    s2m = jnp.where(iota_e == i1, -jnp.inf, s)
    m2 = jnp.max(s2m, axis=1, keepdims=True)
    i2 = jnp.min(jnp.where(s2m == m2, iota_e, E), axis=1, keepdims=True)
    w1 = 1.0 / (1.0 + jnp.exp((m2 - m1) / T))
    w2 = 1.0 - w1

    @pl.when(b == 0)
    def _init():
        carry[...] = jnp.zeros((1, E), jnp.float32)

    one1 = (iota_e == i1).astype(jnp.float32)
    one2 = (iota_e == i2).astype(jnp.float32)
    O = jnp.concatenate([one1, one2], axis=0)
    n2 = 2 * BT
    ir = lax.broadcasted_iota(jnp.int32, (n2, n2), 0)
    ic = lax.broadcasted_iota(jnp.int32, (n2, n2), 1)
    tri = (ic < ir).astype(jnp.float32)
    prior = lax.dot_general(tri, O, (((1,), (0,)), ((), ())),
                            preferred_element_type=jnp.float32)
    cprior = prior + carry[...]
    r1 = jnp.sum(cprior[:BT] * one1, axis=1, keepdims=True)
    r2 = jnp.sum(cprior[BT:] * one2, axis=1, keepdims=True)
    colsum = jnp.sum(O, axis=0, keepdims=True)
    carry[...] += colsum

    sl = pl.ds(b * BT, BT)
    idxs[sl, :] = jnp.concatenate([i1, i2], axis=1)
    w_ref[...] = jnp.concatenate([w1, w2], axis=1)
    ranks[sl, :] = jnp.concatenate([r1, r2], axis=1).astype(jnp.int32)


def _finalize_block(carry, idxs, ranks, dest_ref, bexp_ref):
    tot = carry[...]
    pt = jnp.floor((tot + (BLK - 1)) / BLK) * BLK
    ir8 = lax.broadcasted_iota(jnp.int32, (E, E), 0)
    ic8 = lax.broadcasted_iota(jnp.int32, (E, E), 1)
    ptb = jnp.broadcast_to(pt, (E, E))
    D = jnp.where(ir8 == ic8, ptb, 0.0)
    ptc = lax.dot_general(D, jnp.ones((E, E), jnp.float32),
                          (((1,), (0,)), ((), ())),
                          preferred_element_type=jnp.float32)
    pstart = jnp.sum(jnp.where(ir8 < ic8, ptc, 0.0), axis=0, keepdims=True)

    idx = idxs[...]
    rank = ranks[...]
    for k in range(2):
        ik = idx[:, k:k + 1]
        iota_e = lax.broadcasted_iota(jnp.int32, (B, E), 1)
        onek = (iota_e == ik).astype(jnp.float32)
        base = jnp.sum(onek * pstart, axis=1, keepdims=True)
        dest_ref[:, k:k + 1] = rank[:, k:k + 1] + base.astype(jnp.int32)

    jr = lax.broadcasted_iota(jnp.int32, (NBX, E), 0)
    starts = jnp.broadcast_to(pstart, (NBX, E))
    jstart = (jr * BLK).astype(jnp.float32)
    cnt = jnp.sum((starts <= jstart).astype(jnp.int32), axis=1, keepdims=True)
    total_padded = jnp.sum(pt)
    used = jstart[:, :1] < total_padded
    bexp_ref[...] = jnp.where(used, cnt - 1, -1)


def _gemm_body(s_ref, x_ref, w1_ref, b1_ref, w2_ref, b2_ref, ws_ref, o_ref):
    i = pl.program_id(0)

    @pl.when(s_ref[i] >= 0)
    def _compute():
        x = x_ref[...].astype(jnp.bfloat16)
        w1b = w1_ref[0].astype(jnp.bfloat16)
        hid = jnp.maximum(
            lax.dot_general(x, w1b, (((1,), (0,)), ((), ())),
                            preferred_element_type=jnp.float32) + b1_ref[0], 0.0)
        w2b = w2_ref[0].astype(jnp.bfloat16)
        o = lax.dot_general(hid.astype(jnp.bfloat16), w2b,
                            (((1,), (0,)), ((), ())),
                            preferred_element_type=jnp.float32) + b2_ref[0]
        o_ref[...] = o * ws_ref[:, 0:1]


@functools.lru_cache(maxsize=None)
def _make_dispatch():
    mesh = plsc.VectorSubcoreMesh(core_axis_name="c", subcore_axis_name="s",
                                  num_cores=2)

    @functools.partial(
        pl.kernel, mesh=mesh,
        out_type=[
            jax.ShapeDtypeStruct((P, F), jnp.float32),
            jax.ShapeDtypeStruct((P, 128), jnp.float32),
        ],
        scratch_types=[
            pltpu.VMEM((4, 32), jnp.int32),
            pltpu.VMEM((4, 32), jnp.int32),
            pltpu.VMEM((4, 32, 128), jnp.float32),
            pltpu.VMEM((32, F), jnp.float32),
            pltpu.VMEM((32, F), jnp.float32),
            pltpu.VMEM((32, F), jnp.float32),
            pltpu.SemaphoreType.DMA,
            pltpu.SemaphoreType.DMA,
            pltpu.SemaphoreType.DMA,
            pltpu.SemaphoreType.DMA,
            pltpu.SemaphoreType.DMA,
            pltpu.SemaphoreType.DMA,
            pltpu.SemaphoreType.DMA,
        ],
    )
    def dispatch(h_hbm, tok_hbm, dest_hbm, w_hbm, xs_hbm, ws_hbm, tokv, didv,
                 wv, r0, r1, r2, g0, g1, g2, s0, s1, s2, sw):
        wid = lax.axis_index("s") * 2 + lax.axis_index("c")
        pltpu.sync_copy(tok_hbm.at[wid], tokv)
        pltpu.sync_copy(dest_hbm.at[wid], didv)
        pltpu.sync_copy(w_hbm.at[wid], wv)
        bufs = [r0, r1, r2]
        gsems = [g0, g1, g2]
        ssems = [s0, s1, s2]
        gh = {}
        sh = {}
        wh = {}
        for j in range(3):
            gh[j] = pltpu.async_copy(h_hbm.at[tokv.at[j]], bufs[j], gsems[j])
        for j in range(4):
            p = j % 3
            wh[j] = pltpu.async_copy(wv.at[j], ws_hbm.at[didv.at[j]], sw)
            gh[j].wait()
            sh[j] = pltpu.async_copy(bufs[p], xs_hbm.at[didv.at[j]], ssems[p])
            if j + 3 < 4:
                sh[j].wait()
                gh[j + 3] = pltpu.async_copy(
                    h_hbm.at[tokv.at[j + 3]], bufs[p], gsems[p])
        for j in range(1, 4):
            sh[j].wait()
        for j in range(4):
            wh[j].wait()

    return dispatch


@functools.lru_cache(maxsize=None)
def _make_combine():
    mesh = plsc.VectorSubcoreMesh(core_axis_name="c", subcore_axis_name="s",
                                  num_cores=2)

    @functools.partial(
        pl.kernel, mesh=mesh,
        out_type=jax.ShapeDtypeStruct((B, C), jnp.float32),
        scratch_types=[
            pltpu.VMEM((4, 16), jnp.int32),
            pltpu.VMEM((4, 16), jnp.int32),
            pltpu.VMEM((16, C), jnp.float32),
            pltpu.VMEM((16, C), jnp.float32),
            pltpu.VMEM((16, C), jnp.float32),
            pltpu.VMEM((16, C), jnp.float32),
            pltpu.VMEM((16, C), jnp.float32),
            pltpu.VMEM((16, C), jnp.float32),
            pltpu.SemaphoreType.DMA,
            pltpu.SemaphoreType.DMA,
            pltpu.SemaphoreType.DMA,
            pltpu.SemaphoreType.DMA,
        ],
    )
    def combine(os_hbm, d0_hbm, d1_hbm, out_hbm,
                d0v, d1v, b0a, b0b, b1a, b1b, oa, ob,
                ga, gb, wsa, wsb):
        wid = lax.axis_index("s") * 2 + lax.axis_index("c")
        pltpu.sync_copy(d0_hbm.at[wid], d0v)
        pltpu.sync_copy(d1_hbm.at[wid], d1v)
        b0 = [b0a, b0b]
        b1 = [b1a, b1b]
        ob_ = [oa, ob]
        gsem = [ga, gb]
        wsem = [wsa, wsb]
        gh = {}
        wh = {}
        for j in range(2):
            gh[j] = (pltpu.async_copy(os_hbm.at[d0v.at[j]], b0[j], gsem[j]),
                     pltpu.async_copy(os_hbm.at[d1v.at[j]], b1[j], gsem[j]))
        for j in range(4):
            p = j % 2
            gh[j][0].wait()
            gh[j][1].wait()
            if j >= 2:
                wh[j - 2].wait()

            def body(c, _, p=p):
                for r in range(16):
                    sl = pl.ds(c * 16, 16)
                    ob_[p][r, sl] = b0[p][r, sl] + b1[p][r, sl]
                return 0

            lax.fori_loop(0, C // 16, body, 0)
            wh[j] = pltpu.async_copy(
                ob_[p], out_hbm.at[pl.ds(wid * 64 + j * 16, 16)], wsem[p])
            if j + 2 < 4:
                gh[j + 2] = (
                    pltpu.async_copy(os_hbm.at[d0v.at[j + 2]], b0[p], gsem[p]),
                    pltpu.async_copy(os_hbm.at[d1v.at[j + 2]], b1[p], gsem[p]))
        wh[2].wait()
        wh[3].wait()

    return combine


def kernel(h, W_mu, W_logvar, b_mu, b_logvar, W1, b1, W2, b2):
    bmu = b_mu.reshape(1, E)
    blv = b_logvar.reshape(1, E)
    b1r = b1.reshape(E, 1, H)
    b2r = b2.reshape(E, 1, C)

    wts, dest, bexp = pl.pallas_call(
        _route_body,
        grid=(NB + 1,),
        in_specs=[
            pl.BlockSpec((BT, F), lambda b: (jnp.minimum(b, NB - 1), 0)),
            pl.BlockSpec((E, F), lambda b: (0, 0)),
            pl.BlockSpec((E, F), lambda b: (0, 0)),
            pl.BlockSpec((1, E), lambda b: (0, 0)),
            pl.BlockSpec((1, E), lambda b: (0, 0)),
        ],
        out_specs=[
            pl.BlockSpec((BT, 2), lambda b: (jnp.minimum(b, NB - 1), 0)),
            pl.BlockSpec((B, 2), lambda b: (0, 0)),
            pl.BlockSpec((NBX, 1), lambda b: (0, 0)),
        ],
        out_shape=[
            jax.ShapeDtypeStruct((B, 2), jnp.float32),
            jax.ShapeDtypeStruct((B, 2), jnp.int32),
            jax.ShapeDtypeStruct((NBX, 1), jnp.int32),
        ],
        scratch_shapes=[
            pltpu.VMEM((1, E), jnp.float32),
            pltpu.VMEM((B, 2), jnp.int32),
            pltpu.VMEM((B, 2), jnp.int32),
        ],
        compiler_params=pltpu.CompilerParams(
            dimension_semantics=("arbitrary",)),
    )(h, W_mu, W_logvar, bmu, blv)

    tok3 = jnp.repeat(jnp.arange(B, dtype=jnp.int32), 2).reshape(NW, 4, 32)
    dest3 = dest.reshape(NW, 4, 32)
    wrep = jnp.broadcast_to(wts.reshape(S, 1), (S, 128)).reshape(NW, 4, 32, 128)
    x_sorted, w_sorted = _make_dispatch()(h, tok3, dest3, wrep)

    out_sorted = pl.pallas_call(
        _gemm_body,
        grid_spec=pltpu.PrefetchScalarGridSpec(
            num_scalar_prefetch=1,
            grid=(NBLK,),
            in_specs=[
                pl.BlockSpec((BLK, F), lambda i, s: (i, 0)),
                pl.BlockSpec((1, F, H),
                             lambda i, s: (jnp.where(s[i] < 0, E - 1, s[i]), 0, 0)),
                pl.BlockSpec((1, 1, H),
                             lambda i, s: (jnp.where(s[i] < 0, E - 1, s[i]), 0, 0)),
                pl.BlockSpec((1, H, C),
                             lambda i, s: (jnp.where(s[i] < 0, E - 1, s[i]), 0, 0)),
                pl.BlockSpec((1, 1, C),
                             lambda i, s: (jnp.where(s[i] < 0, E - 1, s[i]), 0, 0)),
                pl.BlockSpec((BLK, 128), lambda i, s: (i, 0)),
            ],
            out_specs=pl.BlockSpec((BLK, C), lambda i, s: (i, 0)),
        ),
        out_shape=jax.ShapeDtypeStruct((P, C), jnp.float32),
        compiler_params=pltpu.CompilerParams(
            dimension_semantics=("arbitrary",)),
    )(bexp.reshape(NBX), x_sorted, W1, b1r, W2, b2r, w_sorted)

    d0 = dest[:, 0].reshape(NW, 4, 16)
    d1 = dest[:, 1].reshape(NW, 4, 16)
    return _make_combine()(out_sorted, d0, d1)
